# Initial kernel scaffold; baseline (speedup 1.0000x reference)
import jax
import jax.numpy as jnp
from jax.experimental import pallas as pl


def kernel(x, edge_index, batch_index, W0, b0, W1, b1, W2, b2, W3, b3, Wo, bo):
    raise NotImplementedError("write your pallas kernel here")


# speedup vs baseline: 14.5683x; 14.5683x over previous
import functools

import jax
import jax.numpy as jnp
from jax import lax
from jax.experimental import pallas as pl
from jax.experimental.pallas import tpu as pltpu
from jax.experimental.pallas import tpu_sc as plsc

N = 10000
NP = 10240
D = 128
G = 16
E = 320000
NW = 32
CHUNK = 128
CPT = 80
EPAD = NW * CPT * CHUNK
NROW = EPAD // CHUNK
NSUB = 16
STRIPE = NP // NSUB
BLK = 256
NBLK = NP // BLK

_mesh = plsc.VectorSubcoreMesh(core_axis_name="c", subcore_axis_name="s")


def _sc_degree(dst2, zeros, ones):
    @functools.partial(
        pl.kernel, mesh=_mesh,
        out_type=jax.ShapeDtypeStruct((2, NP, D), jnp.float32),
        scratch_types=[
            pltpu.VMEM((CPT, CHUNK), jnp.int32),
            pltpu.VMEM((CHUNK, D), jnp.float32),
            pltpu.VMEM_SHARED((NP, D), jnp.float32),
        ],
    )
    def sc_degree_body(dst_h, z_h, o_h, p_h, dstv, onesv, acc):
        c = lax.axis_index("c")
        s = lax.axis_index("s")
        wid = s * 2 + c
        pltpu.sync_copy(z_h.at[pl.ds(s * STRIPE, STRIPE)],
                        acc.at[pl.ds(s * STRIPE, STRIPE)])
        pltpu.sync_copy(o_h, onesv)
        pltpu.sync_copy(dst_h.at[pl.ds(wid * CPT, CPT)], dstv)
        plsc.subcore_barrier()

        def body(j, carry):
            pltpu.sync_copy(onesv, acc.at[dstv.at[j]], add=True)
            return carry

        lax.fori_loop(0, CPT, body, 0)
        plsc.subcore_barrier()
        pltpu.sync_copy(acc.at[pl.ds(s * STRIPE, STRIPE)],
                        p_h.at[c, pl.ds(s * STRIPE, STRIPE)])

    return sc_degree_body(dst2, zeros, ones)


def _sc_message(y, src2, dst2, zeros):
    @functools.partial(
        pl.kernel, mesh=_mesh,
        out_type=jax.ShapeDtypeStruct((2, NP, D), jnp.float32),
        scratch_types=[
            pltpu.VMEM((CPT, CHUNK), jnp.int32),
            pltpu.VMEM((CPT, CHUNK), jnp.int32),
            pltpu.VMEM((CHUNK, D), jnp.float32),
            pltpu.VMEM_SHARED((NP, D), jnp.float32),
            pltpu.SemaphoreType.DMA,
        ],
    )
    def sc_message_body(y_h, src_h, dst_h, z_h, p_h, srcv, dstv, rows, acc, sem):
        c = lax.axis_index("c")
        s = lax.axis_index("s")
        wid = s * 2 + c
        pltpu.sync_copy(z_h.at[pl.ds(s * STRIPE, STRIPE)],
                        acc.at[pl.ds(s * STRIPE, STRIPE)])
        pltpu.sync_copy(src_h.at[pl.ds(wid * CPT, CPT)], srcv)
        pltpu.sync_copy(dst_h.at[pl.ds(wid * CPT, CPT)], dstv)
        plsc.subcore_barrier()

        def body(j, carry):
            pltpu.async_copy(y_h.at[srcv.at[j]], rows, sem).wait()
            pltpu.sync_copy(rows, acc.at[dstv.at[j]], add=True)
            return carry

        lax.fori_loop(0, CPT, body, 0)
        plsc.subcore_barrier()
        pltpu.sync_copy(acc.at[pl.ds(s * STRIPE, STRIPE)],
                        p_h.at[c, pl.ds(s * STRIPE, STRIPE)])

    return sc_message_body(y, src2, dst2, zeros)


def _dinv_col(dp):
    return 1.0 / jnp.sqrt(dp[0, :, :1] + dp[1, :, :1] + 1.0)


def _tc_first(degp, xp, W0):
    def body(dp_ref, x_ref, w_ref, y_ref):
        dinv = _dinv_col(dp_ref[...])
        y_ref[...] = jnp.dot(x_ref[...] * dinv, w_ref[...],
                             preferred_element_type=jnp.float32,
                             precision=lax.Precision.HIGHEST)

    return pl.pallas_call(
        body, grid=(NBLK,),
        in_specs=[
            pl.BlockSpec((2, BLK, D), lambda i: (0, i, 0)),
            pl.BlockSpec((BLK, D), lambda i: (i, 0)),
            pl.BlockSpec((D, D), lambda i: (0, 0)),
        ],
        out_specs=pl.BlockSpec((BLK, D), lambda i: (i, 0)),
        out_shape=jax.ShapeDtypeStruct((NP, D), jnp.float32),
    )(degp, xp, W0)


def _tc_mid(degp, p, y, b, Wn):
    def body(dp_ref, p_ref, y_ref, b_ref, w_ref, o_ref):
        dinv = _dinv_col(dp_ref[...])
        h = jnp.tanh((p_ref[0] + p_ref[1] + y_ref[...]) * dinv + b_ref[...])
        o_ref[...] = jnp.dot(h * dinv, w_ref[...],
                             preferred_element_type=jnp.float32,
                             precision=lax.Precision.HIGHEST)

    return pl.pallas_call(
        body, grid=(NBLK,),
        in_specs=[
            pl.BlockSpec((2, BLK, D), lambda i: (0, i, 0)),
            pl.BlockSpec((2, BLK, D), lambda i: (0, i, 0)),
            pl.BlockSpec((BLK, D), lambda i: (i, 0)),
            pl.BlockSpec((1, D), lambda i: (0, 0)),
            pl.BlockSpec((D, D), lambda i: (0, 0)),
        ],
        out_specs=pl.BlockSpec((BLK, D), lambda i: (i, 0)),
        out_shape=jax.ShapeDtypeStruct((NP, D), jnp.float32),
    )(degp, p, y, b, Wn)


def _tc_pool(degp, p, y, b, oh, Wo, bo):
    def body(dp_ref, p_ref, y_ref, b_ref, oh_ref, wo_ref, bo_ref,
             out_ref, hid_ref, gmax, gsum, cnt):
        i = pl.program_id(0)
        dinv = _dinv_col(dp_ref[...])
        h = jnp.tanh((p_ref[0] + p_ref[1] + y_ref[...]) * dinv + b_ref[...])
        oh = oh_ref[...]

        @pl.when(i == 0)
        def _init():
            gmax[...] = jnp.full((G, D), -jnp.inf, jnp.float32)
            gsum[...] = jnp.zeros((G, D), jnp.float32)
            cnt[...] = jnp.zeros((G, D), jnp.float32)

        gsum[...] += lax.dot_general(oh, h, (((0,), (0,)), ((), ())),
                                     preferred_element_type=jnp.float32,
                                     precision=lax.Precision.HIGHEST)
        cnt[...] += lax.dot_general(oh, jnp.ones((BLK, D), jnp.float32),
                                    (((0,), (0,)), ((), ())),
                                    preferred_element_type=jnp.float32,
                                    precision=lax.Precision.HIGHEST)
        for g in range(G):
            m = oh[:, g:g + 1] > 0.0
            contrib = jnp.max(jnp.where(m, h, -jnp.inf), axis=0, keepdims=True)
            gmax[g:g + 1, :] = jnp.maximum(gmax[g:g + 1, :], contrib)

        @pl.when(i == NBLK - 1)
        def _fin():
            ga = gsum[...] / jnp.maximum(cnt[...], 1.0)
            hid = jnp.concatenate([gmax[...], ga], axis=1)
            hid_ref[...] = hid
            out_ref[...] = jnp.dot(hid, wo_ref[...],
                                   preferred_element_type=jnp.float32,
                                   precision=lax.Precision.HIGHEST) + bo_ref[...]

    return pl.pallas_call(
        body, grid=(NBLK,),
        in_specs=[
            pl.BlockSpec((2, BLK, D), lambda i: (0, i, 0)),
            pl.BlockSpec((2, BLK, D), lambda i: (0, i, 0)),
            pl.BlockSpec((BLK, D), lambda i: (i, 0)),
            pl.BlockSpec((1, D), lambda i: (0, 0)),
            pl.BlockSpec((BLK, G), lambda i: (i, 0)),
            pl.BlockSpec((2 * D, 1), lambda i: (0, 0)),
            pl.BlockSpec((1, 1), lambda i: (0, 0)),
        ],
        out_specs=[
            pl.BlockSpec((G, 1), lambda i: (0, 0)),
            pl.BlockSpec((G, 2 * D), lambda i: (0, 0)),
        ],
        out_shape=[
            jax.ShapeDtypeStruct((G, 1), jnp.float32),
            jax.ShapeDtypeStruct((G, 2 * D), jnp.float32),
        ],
        scratch_shapes=[
            pltpu.VMEM((G, D), jnp.float32),
            pltpu.VMEM((G, D), jnp.float32),
            pltpu.VMEM((G, D), jnp.float32),
        ],
    )(degp, p, y, b, oh, Wo, bo)


def kernel(x, edge_index, batch_index, W0, b0, W1, b1, W2, b2, W3, b3, Wo, bo):
    src = edge_index[0].astype(jnp.int32)
    dst = edge_index[1].astype(jnp.int32)
    npad = EPAD - E
    pad_src = jnp.arange(npad, dtype=jnp.int32) % N
    pad_dst = N + jnp.arange(npad, dtype=jnp.int32) % (NP - N)
    src2 = jnp.concatenate([src, pad_src]).reshape(NROW, CHUNK)
    dst2 = jnp.concatenate([dst, pad_dst]).reshape(NROW, CHUNK)

    xp = jnp.pad(x, ((0, NP - N), (0, 0)))
    zeros = jnp.zeros((NP, D), jnp.float32)
    ones = jnp.ones((CHUNK, D), jnp.float32)
    ohp = jnp.pad(jax.nn.one_hot(batch_index, G, dtype=jnp.float32),
                  ((0, NP - N), (0, 0)))

    degp = _sc_degree(dst2, zeros, ones)
    y = _tc_first(degp, xp, W0)
    for b, Wn in ((b0, W1), (b1, W2), (b2, W3)):
        part = _sc_message(y, src2, dst2, zeros)
        y = _tc_mid(degp, part, y, b.reshape(1, D), Wn)
    part = _sc_message(y, src2, dst2, zeros)
    out, hidden = _tc_pool(degp, part, y, b3.reshape(1, D), ohp,
                           Wo, bo.reshape(1, 1))
    return (out, hidden)


# speedup vs baseline: 17.7507x; 1.2184x over previous
import functools

import jax
import jax.numpy as jnp
from jax import lax
from jax.experimental import pallas as pl
from jax.experimental.pallas import tpu as pltpu
from jax.experimental.pallas import tpu_sc as plsc

N = 10000
NP = 10240
D = 128
G = 16
E = 320000
NW = 32
CHUNK = 128
CPT = 80
HALF = CPT // 2
EPAD = NW * CPT * CHUNK
NROW = EPAD // CHUNK
NSUB = 16
STRIPE = NP // NSUB
BLK = 256
NBLK = NP // BLK

_mesh = plsc.VectorSubcoreMesh(core_axis_name="c", subcore_axis_name="s")


def _sc_degree(dst2, zeros, ones):
    @functools.partial(
        pl.kernel, mesh=_mesh,
        out_type=jax.ShapeDtypeStruct((2, NP, D), jnp.float32),
        scratch_types=[
            pltpu.VMEM((CPT, CHUNK), jnp.int32),
            pltpu.VMEM((CHUNK, D), jnp.float32),
            pltpu.VMEM_SHARED((NP, D), jnp.float32),
        ],
    )
    def sc_degree_body(dst_h, z_h, o_h, p_h, dstv, onesv, acc):
        c = lax.axis_index("c")
        s = lax.axis_index("s")
        wid = s * 2 + c
        pltpu.sync_copy(z_h.at[pl.ds(s * STRIPE, STRIPE)],
                        acc.at[pl.ds(s * STRIPE, STRIPE)])
        pltpu.sync_copy(o_h, onesv)
        pltpu.sync_copy(dst_h.at[pl.ds(wid * CPT, CPT)], dstv)
        plsc.subcore_barrier()

        def body(j, carry):
            pltpu.sync_copy(onesv, acc.at[dstv.at[j]], add=True)
            return carry

        lax.fori_loop(0, CPT, body, 0)
        plsc.subcore_barrier()
        pltpu.sync_copy(acc.at[pl.ds(s * STRIPE, STRIPE)],
                        p_h.at[c, pl.ds(s * STRIPE, STRIPE)])

    return sc_degree_body(dst2, zeros, ones)


def _sc_message(y, src2, dst2, zeros):
    @functools.partial(
        pl.kernel, mesh=_mesh,
        out_type=jax.ShapeDtypeStruct((2, NP, D), jnp.float32),
        scratch_types=[
            pltpu.VMEM((HALF + 8, CHUNK), jnp.int32),
            pltpu.VMEM((HALF, CHUNK), jnp.int32),
            pltpu.VMEM((CHUNK, D), jnp.float32),
            pltpu.VMEM((CHUNK, D), jnp.float32),
            pltpu.VMEM_SHARED((NP, D), jnp.float32),
            pltpu.SemaphoreType.DMA,
            pltpu.SemaphoreType.DMA,
        ],
    )
    def sc_message_body(y_h, src_h, dst_h, z_h, p_h, srcv, dstv,
                        rows0, rows1, acc, sem0, sem1):
        c = lax.axis_index("c")
        s = lax.axis_index("s")
        wid = s * 2 + c
        rows = (rows0, rows1)
        sems = (sem0, sem1)
        pltpu.sync_copy(z_h.at[pl.ds(s * STRIPE, STRIPE)],
                        acc.at[pl.ds(s * STRIPE, STRIPE)])
        pltpu.sync_copy(src_h.at[pl.ds(wid * CPT, HALF + 8)], srcv)
        pltpu.sync_copy(dst_h.at[pl.ds(wid * CPT, HALF)], dstv)
        pltpu.async_copy(y_h.at[srcv.at[0]], rows0, sem0)
        plsc.subcore_barrier()
        pltpu.make_async_copy(y_h.at[srcv.at[0]], rows0, sem0).wait()

        for half in range(2):
            if half == 1:
                pltpu.sync_copy(src_h.at[pl.ds(wid * CPT + HALF, HALF + 8)],
                                srcv)
                pltpu.sync_copy(dst_h.at[pl.ds(wid * CPT + HALF, HALF)],
                                dstv)

            def body(g, carry):
                for b in range(2):
                    r = 2 * g + b
                    pltpu.async_copy(y_h.at[srcv.at[r + 1]], rows[1 - b],
                                     sems[1 - b])
                    pltpu.sync_copy(rows[b], acc.at[dstv.at[r]], add=True)
                    pltpu.make_async_copy(y_h.at[srcv.at[0]], rows[1 - b],
                                          sems[1 - b]).wait()
                return carry

            lax.fori_loop(0, HALF // 2, body, 0)
        plsc.subcore_barrier()
        pltpu.sync_copy(acc.at[pl.ds(s * STRIPE, STRIPE)],
                        p_h.at[c, pl.ds(s * STRIPE, STRIPE)])

    return sc_message_body(y, src2, dst2, zeros)


def _dinv_col(dp):
    return 1.0 / jnp.sqrt(dp[0, :, :1] + dp[1, :, :1] + 1.0)


def _tc_first(degp, xp, W0):
    def body(dp_ref, x_ref, w_ref, y_ref):
        dinv = _dinv_col(dp_ref[...])
        y_ref[...] = jnp.dot(x_ref[...] * dinv, w_ref[...],
                             preferred_element_type=jnp.float32,
                             precision=lax.Precision.HIGHEST)

    return pl.pallas_call(
        body, grid=(NBLK,),
        in_specs=[
            pl.BlockSpec((2, BLK, D), lambda i: (0, i, 0)),
            pl.BlockSpec((BLK, D), lambda i: (i, 0)),
            pl.BlockSpec((D, D), lambda i: (0, 0)),
        ],
        out_specs=pl.BlockSpec((BLK, D), lambda i: (i, 0)),
        out_shape=jax.ShapeDtypeStruct((NP, D), jnp.float32),
    )(degp, xp, W0)


def _tc_mid(degp, p, y, b, Wn):
    def body(dp_ref, p_ref, y_ref, b_ref, w_ref, o_ref):
        dinv = _dinv_col(dp_ref[...])
        h = jnp.tanh((p_ref[0] + p_ref[1] + y_ref[...]) * dinv + b_ref[...])
        o_ref[...] = jnp.dot(h * dinv, w_ref[...],
                             preferred_element_type=jnp.float32,
                             precision=lax.Precision.HIGHEST)

    return pl.pallas_call(
        body, grid=(NBLK,),
        in_specs=[
            pl.BlockSpec((2, BLK, D), lambda i: (0, i, 0)),
            pl.BlockSpec((2, BLK, D), lambda i: (0, i, 0)),
            pl.BlockSpec((BLK, D), lambda i: (i, 0)),
            pl.BlockSpec((1, D), lambda i: (0, 0)),
            pl.BlockSpec((D, D), lambda i: (0, 0)),
        ],
        out_specs=pl.BlockSpec((BLK, D), lambda i: (i, 0)),
        out_shape=jax.ShapeDtypeStruct((NP, D), jnp.float32),
    )(degp, p, y, b, Wn)


def _tc_pool(degp, p, y, b, oh, Wo, bo):
    def body(dp_ref, p_ref, y_ref, b_ref, oh_ref, wo_ref, bo_ref,
             out_ref, hid_ref, gmax, gsum, cnt):
        i = pl.program_id(0)
        dinv = _dinv_col(dp_ref[...])
        h = jnp.tanh((p_ref[0] + p_ref[1] + y_ref[...]) * dinv + b_ref[...])
        oh = oh_ref[...]

        @pl.when(i == 0)
        def _init():
            gmax[...] = jnp.full((G, D), -jnp.inf, jnp.float32)
            gsum[...] = jnp.zeros((G, D), jnp.float32)
            cnt[...] = jnp.zeros((G, D), jnp.float32)

        gsum[...] += lax.dot_general(oh, h, (((0,), (0,)), ((), ())),
                                     preferred_element_type=jnp.float32,
                                     precision=lax.Precision.HIGHEST)
        cnt[...] += lax.dot_general(oh, jnp.ones((BLK, D), jnp.float32),
                                    (((0,), (0,)), ((), ())),
                                    preferred_element_type=jnp.float32,
                                    precision=lax.Precision.HIGHEST)
        for g in range(G):
            m = oh[:, g:g + 1] > 0.0
            contrib = jnp.max(jnp.where(m, h, -jnp.inf), axis=0, keepdims=True)
            gmax[g:g + 1, :] = jnp.maximum(gmax[g:g + 1, :], contrib)

        @pl.when(i == NBLK - 1)
        def _fin():
            ga = gsum[...] / jnp.maximum(cnt[...], 1.0)
            hid = jnp.concatenate([gmax[...], ga], axis=1)
            hid_ref[...] = hid
            out_ref[...] = jnp.dot(hid, wo_ref[...],
                                   preferred_element_type=jnp.float32,
                                   precision=lax.Precision.HIGHEST) + bo_ref[...]

    return pl.pallas_call(
        body, grid=(NBLK,),
        in_specs=[
            pl.BlockSpec((2, BLK, D), lambda i: (0, i, 0)),
            pl.BlockSpec((2, BLK, D), lambda i: (0, i, 0)),
            pl.BlockSpec((BLK, D), lambda i: (i, 0)),
            pl.BlockSpec((1, D), lambda i: (0, 0)),
            pl.BlockSpec((BLK, G), lambda i: (i, 0)),
            pl.BlockSpec((2 * D, 1), lambda i: (0, 0)),
            pl.BlockSpec((1, 1), lambda i: (0, 0)),
        ],
        out_specs=[
            pl.BlockSpec((G, 1), lambda i: (0, 0)),
            pl.BlockSpec((G, 2 * D), lambda i: (0, 0)),
        ],
        out_shape=[
            jax.ShapeDtypeStruct((G, 1), jnp.float32),
            jax.ShapeDtypeStruct((G, 2 * D), jnp.float32),
        ],
        scratch_shapes=[
            pltpu.VMEM((G, D), jnp.float32),
            pltpu.VMEM((G, D), jnp.float32),
            pltpu.VMEM((G, D), jnp.float32),
        ],
    )(degp, p, y, b, oh, Wo, bo)


def kernel(x, edge_index, batch_index, W0, b0, W1, b1, W2, b2, W3, b3, Wo, bo):
    src = edge_index[0].astype(jnp.int32)
    dst = edge_index[1].astype(jnp.int32)
    npad = EPAD - E
    pad_src = jnp.arange(npad, dtype=jnp.int32) % N
    pad_dst = N + jnp.arange(npad, dtype=jnp.int32) % (NP - N)
    src2 = jnp.pad(jnp.concatenate([src, pad_src]).reshape(NROW, CHUNK),
                   ((0, 8), (0, 0)))
    dst2 = jnp.pad(jnp.concatenate([dst, pad_dst]).reshape(NROW, CHUNK),
                   ((0, 8), (0, 0)))

    xp = jnp.pad(x, ((0, NP - N), (0, 0)))
    zeros = jnp.zeros((NP, D), jnp.float32)
    ones = jnp.ones((CHUNK, D), jnp.float32)
    ohp = jnp.pad(jax.nn.one_hot(batch_index, G, dtype=jnp.float32),
                  ((0, NP - N), (0, 0)))

    degp = _sc_degree(dst2, zeros, ones)
    y = _tc_first(degp, xp, W0)
    for b, Wn in ((b0, W1), (b1, W2), (b2, W3)):
        part = _sc_message(y, src2, dst2, zeros)
        y = _tc_mid(degp, part, y, b.reshape(1, D), Wn)
    part = _sc_message(y, src2, dst2, zeros)
    out, hidden = _tc_pool(degp, part, y, b3.reshape(1, D), ohp,
                           Wo, bo.reshape(1, 1))
    return (out, hidden)


# speedup vs baseline: 18.1142x; 1.0205x over previous
import functools

import jax
import jax.numpy as jnp
from jax import lax
from jax.experimental import pallas as pl
from jax.experimental.pallas import tpu as pltpu
from jax.experimental.pallas import tpu_sc as plsc

N = 10000
NP = 10240
D = 128
G = 16
E = 320000
NW = 32
CHUNK = 128
CPT = 80
HALF = CPT // 2
EPAD = NW * CPT * CHUNK
NROW = EPAD // CHUNK
NSUB = 16
STRIPE = NP // NSUB
BLK = 256
NBLK = NP // BLK

_mesh = plsc.VectorSubcoreMesh(core_axis_name="c", subcore_axis_name="s")


def _fill(ref, value):
    v = jnp.full((16,), value, jnp.float32)

    def body(r, carry):
        for k in range(D // 16):
            ref[r, pl.ds(k * 16, 16)] = v
        return carry

    lax.fori_loop(0, CHUNK, body, 0)


def _sc_degree(dst2):
    @functools.partial(
        pl.kernel, mesh=_mesh,
        out_type=jax.ShapeDtypeStruct((2, NP, D), jnp.float32),
        scratch_types=[
            pltpu.VMEM((CPT, CHUNK), jnp.int32),
            pltpu.VMEM((CHUNK, D), jnp.float32),
            pltpu.VMEM_SHARED((NP, D), jnp.float32),
        ],
    )
    def sc_degree_body(dst_h, p_h, dstv, onesv, acc):
        c = lax.axis_index("c")
        s = lax.axis_index("s")
        wid = s * 2 + c
        _fill(onesv, 0.0)
        for k in range(STRIPE // CHUNK):
            pltpu.sync_copy(onesv,
                            acc.at[pl.ds(s * STRIPE + k * CHUNK, CHUNK)])
        _fill(onesv, 1.0)
        pltpu.sync_copy(dst_h.at[pl.ds(wid * CPT, CPT)], dstv)
        plsc.subcore_barrier()

        def body(j, carry):
            pltpu.sync_copy(onesv, acc.at[dstv.at[j]], add=True)
            return carry

        lax.fori_loop(0, CPT, body, 0)
        plsc.subcore_barrier()
        pltpu.sync_copy(acc.at[pl.ds(s * STRIPE, STRIPE)],
                        p_h.at[c, pl.ds(s * STRIPE, STRIPE)])

    return sc_degree_body(dst2)


def _sc_message(y, src2, dst2):
    @functools.partial(
        pl.kernel, mesh=_mesh,
        out_type=jax.ShapeDtypeStruct((2, NP, D), jnp.float32),
        scratch_types=[
            pltpu.VMEM((HALF + 8, CHUNK), jnp.int32),
            pltpu.VMEM((HALF, CHUNK), jnp.int32),
            pltpu.VMEM((CHUNK, D), jnp.float32),
            pltpu.VMEM((CHUNK, D), jnp.float32),
            pltpu.VMEM_SHARED((NP, D), jnp.float32),
            pltpu.SemaphoreType.DMA,
            pltpu.SemaphoreType.DMA,
        ],
    )
    def sc_message_body(y_h, src_h, dst_h, p_h, srcv, dstv,
                        rows0, rows1, acc, sem0, sem1):
        c = lax.axis_index("c")
        s = lax.axis_index("s")
        wid = s * 2 + c
        rows = (rows0, rows1)
        sems = (sem0, sem1)
        _fill(rows0, 0.0)
        for k in range(STRIPE // CHUNK):
            pltpu.sync_copy(rows0,
                            acc.at[pl.ds(s * STRIPE + k * CHUNK, CHUNK)])
        pltpu.sync_copy(src_h.at[pl.ds(wid * CPT, HALF + 8)], srcv)
        pltpu.sync_copy(dst_h.at[pl.ds(wid * CPT, HALF)], dstv)
        pltpu.async_copy(y_h.at[srcv.at[0]], rows0, sem0)
        plsc.subcore_barrier()
        pltpu.make_async_copy(y_h.at[srcv.at[0]], rows0, sem0).wait()

        for half in range(2):
            if half == 1:
                pltpu.sync_copy(src_h.at[pl.ds(wid * CPT + HALF, HALF + 8)],
                                srcv)
                pltpu.sync_copy(dst_h.at[pl.ds(wid * CPT + HALF, HALF)],
                                dstv)

            def body(g, carry):
                for b in range(2):
                    r = 2 * g + b
                    pltpu.async_copy(y_h.at[srcv.at[r + 1]], rows[1 - b],
                                     sems[1 - b])
                    pltpu.sync_copy(rows[b], acc.at[dstv.at[r]], add=True)
                    pltpu.make_async_copy(y_h.at[srcv.at[0]], rows[1 - b],
                                          sems[1 - b]).wait()
                return carry

            lax.fori_loop(0, HALF // 2, body, 0)
        plsc.subcore_barrier()
        pltpu.sync_copy(acc.at[pl.ds(s * STRIPE, STRIPE)],
                        p_h.at[c, pl.ds(s * STRIPE, STRIPE)])

    return sc_message_body(y, src2, dst2)


def _dinv_col(dp):
    return 1.0 / jnp.sqrt(dp[0, :, :1] + dp[1, :, :1] + 1.0)


def _tc_first(degp, xp, W0):
    def body(dp_ref, x_ref, w_ref, y_ref):
        dinv = _dinv_col(dp_ref[...])
        y_ref[...] = jnp.dot(x_ref[...] * dinv, w_ref[...],
                             preferred_element_type=jnp.float32,
                             precision=lax.Precision.HIGHEST)

    return pl.pallas_call(
        body, grid=(NBLK,),
        in_specs=[
            pl.BlockSpec((2, BLK, D), lambda i: (0, i, 0)),
            pl.BlockSpec((BLK, D), lambda i: (i, 0)),
            pl.BlockSpec((D, D), lambda i: (0, 0)),
        ],
        out_specs=pl.BlockSpec((BLK, D), lambda i: (i, 0)),
        out_shape=jax.ShapeDtypeStruct((NP, D), jnp.float32),
    )(degp, xp, W0)


def _tc_mid(degp, p, y, b, Wn):
    def body(dp_ref, p_ref, y_ref, b_ref, w_ref, o_ref):
        dinv = _dinv_col(dp_ref[...])
        h = jnp.tanh((p_ref[0] + p_ref[1] + y_ref[...]) * dinv + b_ref[...])
        o_ref[...] = jnp.dot(h * dinv, w_ref[...],
                             preferred_element_type=jnp.float32,
                             precision=lax.Precision.HIGHEST)

    return pl.pallas_call(
        body, grid=(NBLK,),
        in_specs=[
            pl.BlockSpec((2, BLK, D), lambda i: (0, i, 0)),
            pl.BlockSpec((2, BLK, D), lambda i: (0, i, 0)),
            pl.BlockSpec((BLK, D), lambda i: (i, 0)),
            pl.BlockSpec((1, D), lambda i: (0, 0)),
            pl.BlockSpec((D, D), lambda i: (0, 0)),
        ],
        out_specs=pl.BlockSpec((BLK, D), lambda i: (i, 0)),
        out_shape=jax.ShapeDtypeStruct((NP, D), jnp.float32),
    )(degp, p, y, b, Wn)


def _tc_pool(degp, p, y, b, oh, Wo, bo):
    def body(dp_ref, p_ref, y_ref, b_ref, oh_ref, wo_ref, bo_ref,
             out_ref, hid_ref, gmax, gsum, cnt):
        i = pl.program_id(0)
        dinv = _dinv_col(dp_ref[...])
        h = jnp.tanh((p_ref[0] + p_ref[1] + y_ref[...]) * dinv + b_ref[...])
        oh = oh_ref[...]

        @pl.when(i == 0)
        def _init():
            gmax[...] = jnp.full((G, D), -jnp.inf, jnp.float32)
            gsum[...] = jnp.zeros((G, D), jnp.float32)
            cnt[...] = jnp.zeros((G, D), jnp.float32)

        gsum[...] += lax.dot_general(oh, h, (((0,), (0,)), ((), ())),
                                     preferred_element_type=jnp.float32,
                                     precision=lax.Precision.HIGHEST)
        cnt[...] += lax.dot_general(oh, jnp.ones((BLK, D), jnp.float32),
                                    (((0,), (0,)), ((), ())),
                                    preferred_element_type=jnp.float32,
                                    precision=lax.Precision.HIGHEST)
        for g in range(G):
            m = oh[:, g:g + 1] > 0.0
            contrib = jnp.max(jnp.where(m, h, -jnp.inf), axis=0, keepdims=True)
            gmax[g:g + 1, :] = jnp.maximum(gmax[g:g + 1, :], contrib)

        @pl.when(i == NBLK - 1)
        def _fin():
            ga = gsum[...] / jnp.maximum(cnt[...], 1.0)
            hid = jnp.concatenate([gmax[...], ga], axis=1)
            hid_ref[...] = hid
            out_ref[...] = jnp.dot(hid, wo_ref[...],
                                   preferred_element_type=jnp.float32,
                                   precision=lax.Precision.HIGHEST) + bo_ref[...]

    return pl.pallas_call(
        body, grid=(NBLK,),
        in_specs=[
            pl.BlockSpec((2, BLK, D), lambda i: (0, i, 0)),
            pl.BlockSpec((2, BLK, D), lambda i: (0, i, 0)),
            pl.BlockSpec((BLK, D), lambda i: (i, 0)),
            pl.BlockSpec((1, D), lambda i: (0, 0)),
            pl.BlockSpec((BLK, G), lambda i: (i, 0)),
            pl.BlockSpec((2 * D, 1), lambda i: (0, 0)),
            pl.BlockSpec((1, 1), lambda i: (0, 0)),
        ],
        out_specs=[
            pl.BlockSpec((G, 1), lambda i: (0, 0)),
            pl.BlockSpec((G, 2 * D), lambda i: (0, 0)),
        ],
        out_shape=[
            jax.ShapeDtypeStruct((G, 1), jnp.float32),
            jax.ShapeDtypeStruct((G, 2 * D), jnp.float32),
        ],
        scratch_shapes=[
            pltpu.VMEM((G, D), jnp.float32),
            pltpu.VMEM((G, D), jnp.float32),
            pltpu.VMEM((G, D), jnp.float32),
        ],
    )(degp, p, y, b, oh, Wo, bo)


def kernel(x, edge_index, batch_index, W0, b0, W1, b1, W2, b2, W3, b3, Wo, bo):
    src = edge_index[0].astype(jnp.int32)
    dst = edge_index[1].astype(jnp.int32)
    npad = EPAD - E
    pad_src = jnp.arange(npad, dtype=jnp.int32) % N
    pad_dst = N + jnp.arange(npad, dtype=jnp.int32) % (NP - N)
    src2 = jnp.pad(jnp.concatenate([src, pad_src]).reshape(NROW, CHUNK),
                   ((0, 8), (0, 0)))
    dst2 = jnp.pad(jnp.concatenate([dst, pad_dst]).reshape(NROW, CHUNK),
                   ((0, 8), (0, 0)))

    xp = jnp.pad(x, ((0, NP - N), (0, 0)))
    ohp = jnp.pad(jax.nn.one_hot(batch_index, G, dtype=jnp.float32),
                  ((0, NP - N), (0, 0)))

    degp = _sc_degree(dst2)
    y = _tc_first(degp, xp, W0)
    for b, Wn in ((b0, W1), (b1, W2), (b2, W3)):
        part = _sc_message(y, src2, dst2)
        y = _tc_mid(degp, part, y, b.reshape(1, D), Wn)
    part = _sc_message(y, src2, dst2)
    out, hidden = _tc_pool(degp, part, y, b3.reshape(1, D), ohp,
                           Wo, bo.reshape(1, 1))
    return (out, hidden)


# speedup vs baseline: 18.3698x; 1.0141x over previous
import functools

import jax
import jax.numpy as jnp
from jax import lax
from jax.experimental import pallas as pl
from jax.experimental.pallas import tpu as pltpu
from jax.experimental.pallas import tpu_sc as plsc

N = 10000
NP = 10240
D = 128
G = 16
E = 320000
NW = 32
CHUNK = 128
CPT = 80
HALF = CPT // 2
EPAD = NW * CPT * CHUNK
NROW = EPAD // CHUNK
NSUB = 16
STRIPE = NP // NSUB
BLK = 256
NBLK = NP // BLK

_mesh = plsc.VectorSubcoreMesh(core_axis_name="c", subcore_axis_name="s")


def _fill(ref, value):
    v = jnp.full((16,), value, jnp.float32)

    def body(r, carry):
        for k in range(D // 16):
            ref[r, pl.ds(k * 16, 16)] = v
        return carry

    lax.fori_loop(0, CHUNK, body, 0)


def _sc_degree(dst2):
    @functools.partial(
        pl.kernel, mesh=_mesh,
        out_type=jax.ShapeDtypeStruct((2, NP, D), jnp.float32),
        scratch_types=[
            pltpu.VMEM((CPT, CHUNK), jnp.int32),
            pltpu.VMEM((CHUNK, D), jnp.float32),
            pltpu.VMEM_SHARED((NP, D), jnp.float32),
        ],
    )
    def sc_degree_body(dst_h, p_h, dstv, onesv, acc):
        c = lax.axis_index("c")
        s = lax.axis_index("s")
        wid = s * 2 + c
        _fill(onesv, 0.0)
        for k in range(STRIPE // CHUNK):
            pltpu.sync_copy(onesv,
                            acc.at[pl.ds(s * STRIPE + k * CHUNK, CHUNK)])
        _fill(onesv, 1.0)
        pltpu.sync_copy(dst_h.at[pl.ds(wid * CPT, CPT)], dstv)
        plsc.subcore_barrier()

        def body(j, carry):
            pltpu.sync_copy(onesv, acc.at[dstv.at[j]], add=True)
            return carry

        lax.fori_loop(0, CPT, body, 0)
        plsc.subcore_barrier()
        pltpu.sync_copy(acc.at[pl.ds(s * STRIPE, STRIPE)],
                        p_h.at[c, pl.ds(s * STRIPE, STRIPE)])

    return sc_degree_body(dst2)


def _sc_message(y, src2, dst2):
    @functools.partial(
        pl.kernel, mesh=_mesh,
        out_type=jax.ShapeDtypeStruct((2, NP, D), jnp.float32),
        scratch_types=[
            pltpu.VMEM((HALF + 8, CHUNK), jnp.int32),
            pltpu.VMEM((HALF, CHUNK), jnp.int32),
            pltpu.VMEM((CHUNK, D), jnp.float32),
            pltpu.VMEM((CHUNK, D), jnp.float32),
            pltpu.VMEM_SHARED((NP, D), jnp.float32),
            pltpu.SemaphoreType.DMA,
            pltpu.SemaphoreType.DMA,
        ],
    )
    def sc_message_body(y_h, src_h, dst_h, p_h, srcv, dstv,
                        rows0, rows1, acc, sem0, sem1):
        c = lax.axis_index("c")
        s = lax.axis_index("s")
        wid = s * 2 + c
        rows = (rows0, rows1)
        sems = (sem0, sem1)
        _fill(rows0, 0.0)
        for k in range(STRIPE // CHUNK):
            pltpu.sync_copy(rows0,
                            acc.at[pl.ds(s * STRIPE + k * CHUNK, CHUNK)])
        pltpu.sync_copy(src_h.at[pl.ds(wid * CPT, HALF + 8)], srcv)
        pltpu.sync_copy(dst_h.at[pl.ds(wid * CPT, HALF)], dstv)
        pltpu.async_copy(y_h.at[srcv.at[0]], rows0, sem0)
        plsc.subcore_barrier()
        pltpu.make_async_copy(y_h.at[srcv.at[0]], rows0, sem0).wait()

        for half in range(2):
            if half == 1:
                pltpu.sync_copy(src_h.at[pl.ds(wid * CPT + HALF, HALF + 8)],
                                srcv)
                pltpu.sync_copy(dst_h.at[pl.ds(wid * CPT + HALF, HALF)],
                                dstv)

            def body(g, carry):
                for b in range(2):
                    r = 2 * g + b
                    pltpu.async_copy(y_h.at[srcv.at[r + 1]], rows[1 - b],
                                     sems[1 - b])
                    pltpu.sync_copy(rows[b], acc.at[dstv.at[r]], add=True)
                    pltpu.make_async_copy(y_h.at[srcv.at[0]], rows[1 - b],
                                          sems[1 - b]).wait()
                return carry

            lax.fori_loop(0, HALF // 2, body, 0)
        plsc.subcore_barrier()
        pltpu.sync_copy(acc.at[pl.ds(s * STRIPE, STRIPE)],
                        p_h.at[c, pl.ds(s * STRIPE, STRIPE)])

    return sc_message_body(y, src2, dst2)


def _dinv_col(dp):
    return 1.0 / jnp.sqrt(dp[0, :, :1] + dp[1, :, :1] + 1.0)


def _tc_first(degp, xp, W0):
    def body(dp_ref, x_ref, w_ref, y_ref):
        dinv = _dinv_col(dp_ref[...])
        y_ref[...] = dinv * jnp.dot(x_ref[...], w_ref[...],
                                    preferred_element_type=jnp.float32)

    return pl.pallas_call(
        body, grid=(NBLK,),
        in_specs=[
            pl.BlockSpec((2, BLK, D), lambda i: (0, i, 0)),
            pl.BlockSpec((BLK, D), lambda i: (i, 0)),
            pl.BlockSpec((D, D), lambda i: (0, 0)),
        ],
        out_specs=pl.BlockSpec((BLK, D), lambda i: (i, 0)),
        out_shape=jax.ShapeDtypeStruct((NP, D), jnp.float32),
    )(degp, xp, W0)


def _tc_mid(degp, p, y, b, Wn):
    def body(dp_ref, p_ref, y_ref, b_ref, w_ref, o_ref):
        dinv = _dinv_col(dp_ref[...])
        h = jnp.tanh((p_ref[0] + p_ref[1] + y_ref[...]) * dinv + b_ref[...])
        o_ref[...] = dinv * jnp.dot(h, w_ref[...],
                                    preferred_element_type=jnp.float32)

    return pl.pallas_call(
        body, grid=(NBLK,),
        in_specs=[
            pl.BlockSpec((2, BLK, D), lambda i: (0, i, 0)),
            pl.BlockSpec((2, BLK, D), lambda i: (0, i, 0)),
            pl.BlockSpec((BLK, D), lambda i: (i, 0)),
            pl.BlockSpec((1, D), lambda i: (0, 0)),
            pl.BlockSpec((D, D), lambda i: (0, 0)),
        ],
        out_specs=pl.BlockSpec((BLK, D), lambda i: (i, 0)),
        out_shape=jax.ShapeDtypeStruct((NP, D), jnp.float32),
    )(degp, p, y, b, Wn)


def _tc_pool(degp, p, y, b, oh, Wo, bo):
    def body(dp_ref, p_ref, y_ref, b_ref, oh_ref, wo_ref, bo_ref,
             out_ref, hid_ref, gmax, gsum, cnt):
        i = pl.program_id(0)
        dinv = _dinv_col(dp_ref[...])
        h = jnp.tanh((p_ref[0] + p_ref[1] + y_ref[...]) * dinv + b_ref[...])
        oh = oh_ref[...]

        @pl.when(i == 0)
        def _init():
            gmax[...] = jnp.full((G, D), -jnp.inf, jnp.float32)
            gsum[...] = jnp.zeros((G, D), jnp.float32)
            cnt[...] = jnp.zeros((G, D), jnp.float32)

        gsum[...] += lax.dot_general(oh, h, (((0,), (0,)), ((), ())),
                                     preferred_element_type=jnp.float32,
                                     precision=lax.Precision.HIGHEST)
        cnt[...] += lax.dot_general(oh, jnp.ones((BLK, D), jnp.float32),
                                    (((0,), (0,)), ((), ())),
                                    preferred_element_type=jnp.float32,
                                    precision=lax.Precision.HIGHEST)
        for g in range(G):
            m = oh[:, g:g + 1] > 0.0
            contrib = jnp.max(jnp.where(m, h, -jnp.inf), axis=0, keepdims=True)
            gmax[g:g + 1, :] = jnp.maximum(gmax[g:g + 1, :], contrib)

        @pl.when(i == NBLK - 1)
        def _fin():
            ga = gsum[...] / jnp.maximum(cnt[...], 1.0)
            hid = jnp.concatenate([gmax[...], ga], axis=1)
            hid_ref[...] = hid
            out_ref[...] = jnp.dot(hid, wo_ref[...],
                                   preferred_element_type=jnp.float32) + bo_ref[...]

    return pl.pallas_call(
        body, grid=(NBLK,),
        in_specs=[
            pl.BlockSpec((2, BLK, D), lambda i: (0, i, 0)),
            pl.BlockSpec((2, BLK, D), lambda i: (0, i, 0)),
            pl.BlockSpec((BLK, D), lambda i: (i, 0)),
            pl.BlockSpec((1, D), lambda i: (0, 0)),
            pl.BlockSpec((BLK, G), lambda i: (i, 0)),
            pl.BlockSpec((2 * D, 1), lambda i: (0, 0)),
            pl.BlockSpec((1, 1), lambda i: (0, 0)),
        ],
        out_specs=[
            pl.BlockSpec((G, 1), lambda i: (0, 0)),
            pl.BlockSpec((G, 2 * D), lambda i: (0, 0)),
        ],
        out_shape=[
            jax.ShapeDtypeStruct((G, 1), jnp.float32),
            jax.ShapeDtypeStruct((G, 2 * D), jnp.float32),
        ],
        scratch_shapes=[
            pltpu.VMEM((G, D), jnp.float32),
            pltpu.VMEM((G, D), jnp.float32),
            pltpu.VMEM((G, D), jnp.float32),
        ],
    )(degp, p, y, b, oh, Wo, bo)


def kernel(x, edge_index, batch_index, W0, b0, W1, b1, W2, b2, W3, b3, Wo, bo):
    src = edge_index[0].astype(jnp.int32)
    dst = edge_index[1].astype(jnp.int32)
    npad = EPAD - E
    pad_src = jnp.arange(npad, dtype=jnp.int32) % N
    pad_dst = N + jnp.arange(npad, dtype=jnp.int32) % (NP - N)
    src2 = jnp.pad(jnp.concatenate([src, pad_src]).reshape(NROW, CHUNK),
                   ((0, 8), (0, 0)))
    dst2 = jnp.pad(jnp.concatenate([dst, pad_dst]).reshape(NROW, CHUNK),
                   ((0, 8), (0, 0)))

    xp = jnp.pad(x, ((0, NP - N), (0, 0)))
    ohp = jnp.pad(jax.nn.one_hot(batch_index, G, dtype=jnp.float32),
                  ((0, NP - N), (0, 0)))

    degp = _sc_degree(dst2)
    y = _tc_first(degp, xp, W0)
    for b, Wn in ((b0, W1), (b1, W2), (b2, W3)):
        part = _sc_message(y, src2, dst2)
        y = _tc_mid(degp, part, y, b.reshape(1, D), Wn)
    part = _sc_message(y, src2, dst2)
    out, hidden = _tc_pool(degp, part, y, b3.reshape(1, D), ohp,
                           Wo, bo.reshape(1, 1))
    return (out, hidden)
